# baseline (speedup 1.0000x reference)
import functools

import jax
import jax.numpy as jnp
from jax import lax
from jax.experimental import pallas as pl
from jax.experimental.pallas import tpu as pltpu
from jax.experimental.pallas import tpu_sc as plsc

N_LATENT = 128
N_GENES = 100000
N_OUT = 16
N_HIDDEN = 16
BATCH = 1024
G_OI = 2048

_NW = 32
_WROWS = G_OI * N_HIDDEN // _NW
_WCHUNK = 128
_NCHUNK = _WROWS // _WCHUNK
_BROWS = G_OI // _NW


def _sc_gather_body(idx_hbm, tbl_hbm, genes_hbm, btbl_hbm,
                    wt_hbm, bg_hbm,
                    idx_v, rows_v, bidx_v, brows_v, sem, bsem):
    wid = lax.axis_index("s") * 2 + lax.axis_index("c")
    pltpu.sync_copy(idx_hbm.at[pl.ds(wid * _NCHUNK, _NCHUNK)], idx_v)
    copies = [
        pltpu.async_copy(tbl_hbm.at[idx_v.at[j]],
                         rows_v.at[pl.ds(j * _WCHUNK, _WCHUNK)], sem)
        for j in range(_NCHUNK)
    ]
    pltpu.sync_copy(genes_hbm.at[pl.ds(wid * _BROWS, _BROWS)], bidx_v)
    bcopy = pltpu.async_copy(btbl_hbm.at[bidx_v], brows_v, bsem)
    for c in copies:
        c.wait()
    pltpu.sync_copy(rows_v, wt_hbm.at[pl.ds(wid * _WROWS, _WROWS)])
    bcopy.wait()
    pltpu.sync_copy(brows_v, bg_hbm.at[pl.ds(wid * _BROWS, _BROWS)])


@functools.partial(jax.jit, static_argnames=())
def _sc_gather(idx, tbl_fine, genes, btbl):
    mesh = plsc.VectorSubcoreMesh(core_axis_name="c", subcore_axis_name="s")
    return pl.kernel(
        _sc_gather_body,
        out_type=(
            jax.ShapeDtypeStruct((G_OI * N_HIDDEN, N_OUT), jnp.float32),
            jax.ShapeDtypeStruct((G_OI, N_OUT), jnp.float32),
        ),
        mesh=mesh,
        scratch_types=(
            pltpu.VMEM((_NCHUNK, _WCHUNK), jnp.int32),
            pltpu.VMEM((_WROWS, N_OUT), jnp.float32),
            pltpu.VMEM((_BROWS,), jnp.int32),
            pltpu.VMEM((_BROWS, N_OUT), jnp.float32),
            pltpu.SemaphoreType.DMA,
            pltpu.SemaphoreType.DMA,
        ),
        compiler_params=pltpu.CompilerParams(use_tc_tiling_on_sc=False),
    )(idx, tbl_fine, genes, btbl)


def _tc_matmul_body(latent_ref, w1_ref, b1_ref, scale_ref, shift_ref,
                    wt_ref, bias_ref, out_ref):
    h = jnp.dot(latent_ref[...], w1_ref[...], preferred_element_type=jnp.float32)
    h = jnp.maximum(h + b1_ref[...], 0.0)
    h = h * scale_ref[...] + shift_ref[...]
    out_ref[...] = (
        jnp.dot(h, wt_ref[...], preferred_element_type=jnp.float32)
        + bias_ref[...]
    )


_BB = 256
_GBC = 8192


def _tc_matmul(latent, w1, b1, scale, shift, wt, bias_flat):
    ncols = G_OI * N_OUT
    grid = (BATCH // _BB, ncols // _GBC)
    return pl.pallas_call(
        _tc_matmul_body,
        grid=grid,
        in_specs=[
            pl.BlockSpec((_BB, N_LATENT), lambda i, j: (i, 0)),
            pl.BlockSpec((N_LATENT, N_HIDDEN), lambda i, j: (0, 0)),
            pl.BlockSpec((1, N_HIDDEN), lambda i, j: (0, 0)),
            pl.BlockSpec((1, N_HIDDEN), lambda i, j: (0, 0)),
            pl.BlockSpec((1, N_HIDDEN), lambda i, j: (0, 0)),
            pl.BlockSpec((N_HIDDEN, _GBC), lambda i, j: (0, j)),
            pl.BlockSpec((1, _GBC), lambda i, j: (0, j)),
        ],
        out_specs=pl.BlockSpec((_BB, _GBC), lambda i, j: (i, j)),
        out_shape=jax.ShapeDtypeStruct((BATCH, ncols), jnp.float32),
        compiler_params=pltpu.CompilerParams(
            dimension_semantics=("parallel", "parallel"),
        ),
    )(latent, w1, b1, scale, shift, wt, bias_flat)


def kernel(latent, genes_oi, W1, b1, gamma, beta, run_mean, run_var,
           weight_table, bias_table):
    genes = genes_oi.astype(jnp.int32)
    idx = (genes[None, :] * N_HIDDEN
           + jnp.arange(N_HIDDEN, dtype=jnp.int32)[:, None])
    idx = idx.reshape(_NW * _NCHUNK, _WCHUNK)
    tbl_fine = weight_table.reshape(N_GENES * N_HIDDEN, N_OUT)

    wt_rows, b_g = _sc_gather(idx, tbl_fine, genes, bias_table)
    wt = wt_rows.reshape(N_HIDDEN, G_OI * N_OUT)
    bias_flat = b_g.reshape(1, G_OI * N_OUT)

    scale = (gamma / jnp.sqrt(run_var + 1e-5)).reshape(1, N_HIDDEN)
    shift = (beta - run_mean * scale[0]).reshape(1, N_HIDDEN)

    out2d = _tc_matmul(latent, W1, b1.reshape(1, N_HIDDEN), scale, shift,
                       wt, bias_flat)
    return out2d.reshape(BATCH, G_OI, N_OUT)


# speedup vs baseline: 2.4828x; 2.4828x over previous
import jax
import jax.numpy as jnp
from jax import lax
from jax.experimental import pallas as pl
from jax.experimental.pallas import tpu as pltpu
from jax.experimental.pallas import tpu_sc as plsc

N_LATENT = 128
N_GENES = 100000
N_OUT = 16
N_HIDDEN = 16
BATCH = 1024
G_OI = 2048

_NW = 32
_GPW = G_OI // _NW
_CPW = _GPW * N_OUT
_NCOLS = G_OI * N_OUT


def _sc_gather_body(genes_hbm, wtbl_hbm, bri_hbm, cmap_hbm, btbl_hbm,
                    wt_hbm, bflat_hbm,
                    idx_v, bri_v, cmap_v, rows_v, brow_v, wtile, btile,
                    wsem, bsem):
    wid = lax.axis_index("s") * 2 + lax.axis_index("c")
    base = wid * _GPW
    pltpu.sync_copy(genes_hbm.at[pl.ds(base, _GPW)], idx_v)
    wcopy = pltpu.async_copy(wtbl_hbm.at[idx_v], rows_v, wsem)
    pltpu.sync_copy(bri_hbm.at[pl.ds(base, _GPW)], bri_v)
    bcopy = pltpu.async_copy(btbl_hbm.at[bri_v], brow_v, bsem)
    pltpu.sync_copy(cmap_hbm.at[pl.ds(base, _GPW)], cmap_v)

    wcopy.wait()

    def transpose_one(g, carry):
        for h in range(N_HIDDEN):
            wtile[h, pl.ds(g * N_OUT, N_OUT)] = rows_v[g, pl.ds(h * N_OUT, N_OUT)]
        return carry

    lax.fori_loop(0, _GPW, transpose_one, 0, unroll=False)
    pltpu.sync_copy(wtile, wt_hbm.at[:, pl.ds(wid * _CPW, _CPW)])

    bcopy.wait()

    def extract_bias(g, carry):
        base = cmap_v[g, :] - lax.iota(jnp.int32, 16)
        acc = jnp.zeros((16,), dtype=jnp.float32)
        for k in range(8):
            chunk = brow_v[g, pl.ds(k * N_OUT, N_OUT)]
            acc = jnp.where(base == k * N_OUT, chunk, acc)
        btile[0, pl.ds(g * N_OUT, N_OUT)] = acc
        return carry

    lax.fori_loop(0, _GPW, extract_bias, 0, unroll=False)
    pltpu.sync_copy(btile, bflat_hbm.at[:, pl.ds(wid * _CPW, _CPW)])


def _sc_gather(genes, wtbl256, bri, cmap, btbl128):
    mesh = plsc.VectorSubcoreMesh(core_axis_name="c", subcore_axis_name="s")
    return pl.kernel(
        _sc_gather_body,
        out_type=(
            jax.ShapeDtypeStruct((N_HIDDEN, _NCOLS), jnp.float32),
            jax.ShapeDtypeStruct((1, _NCOLS), jnp.float32),
        ),
        mesh=mesh,
        scratch_types=(
            pltpu.VMEM((_GPW,), jnp.int32),
            pltpu.VMEM((_GPW,), jnp.int32),
            pltpu.VMEM((_GPW, N_OUT), jnp.int32),
            pltpu.VMEM((_GPW, N_HIDDEN * N_OUT), jnp.float32),
            pltpu.VMEM((_GPW, 128), jnp.float32),
            pltpu.VMEM((N_HIDDEN, _CPW), jnp.float32),
            pltpu.VMEM((1, _CPW), jnp.float32),
            pltpu.SemaphoreType.DMA,
            pltpu.SemaphoreType.DMA,
        ),
    )(genes, wtbl256, bri, cmap, btbl128)


def _tc_matmul_body(latent_ref, w1_ref, b1_ref, scale_ref, shift_ref,
                    wt_ref, bias_ref, out_ref):
    h = jnp.dot(latent_ref[...], w1_ref[...], preferred_element_type=jnp.float32)
    h = jnp.maximum(h + b1_ref[...], 0.0)
    h = h * scale_ref[...] + shift_ref[...]
    out_ref[...] = (
        jnp.dot(h, wt_ref[...], preferred_element_type=jnp.float32)
        + bias_ref[...]
    )


_BB = 256
_GBC = 8192


def _tc_matmul(latent, w1, b1, scale, shift, wt, bias_flat):
    grid = (BATCH // _BB, _NCOLS // _GBC)
    return pl.pallas_call(
        _tc_matmul_body,
        grid=grid,
        in_specs=[
            pl.BlockSpec((_BB, N_LATENT), lambda i, j: (i, 0)),
            pl.BlockSpec((N_LATENT, N_HIDDEN), lambda i, j: (0, 0)),
            pl.BlockSpec((1, N_HIDDEN), lambda i, j: (0, 0)),
            pl.BlockSpec((1, N_HIDDEN), lambda i, j: (0, 0)),
            pl.BlockSpec((1, N_HIDDEN), lambda i, j: (0, 0)),
            pl.BlockSpec((N_HIDDEN, _GBC), lambda i, j: (0, j)),
            pl.BlockSpec((1, _GBC), lambda i, j: (0, j)),
        ],
        out_specs=pl.BlockSpec((_BB, _GBC), lambda i, j: (i, j)),
        out_shape=jax.ShapeDtypeStruct((BATCH, _NCOLS), jnp.float32),
        compiler_params=pltpu.CompilerParams(
            dimension_semantics=("parallel", "parallel"),
        ),
    )(latent, w1, b1, scale, shift, wt, bias_flat)


def kernel(latent, genes_oi, W1, b1, gamma, beta, run_mean, run_var,
           weight_table, bias_table):
    genes = genes_oi.astype(jnp.int32)
    wtbl256 = weight_table.reshape(N_GENES, N_HIDDEN * N_OUT)
    btbl128 = bias_table.reshape(N_GENES * N_OUT // 128, 128)
    bri = genes // 8
    cmap = ((genes % 8) * N_OUT)[:, None] + jnp.arange(N_OUT, dtype=jnp.int32)[None, :]

    wt, bias_flat = _sc_gather(genes, wtbl256, bri, cmap, btbl128)

    scale = (gamma / jnp.sqrt(run_var + 1e-5)).reshape(1, N_HIDDEN)
    shift = (beta - run_mean * scale[0]).reshape(1, N_HIDDEN)

    out2d = _tc_matmul(latent, W1, b1.reshape(1, N_HIDDEN), scale, shift,
                       wt, bias_flat)
    return out2d.reshape(BATCH, G_OI, N_OUT)


# speedup vs baseline: 3.4318x; 1.3822x over previous
import jax
import jax.numpy as jnp
from jax import lax
from jax.experimental import pallas as pl
from jax.experimental.pallas import tpu as pltpu
from jax.experimental.pallas import tpu_sc as plsc

N_LATENT = 128
N_GENES = 100000
N_OUT = 16
N_HIDDEN = 16
BATCH = 1024
G_OI = 2048

_NW = 32
_GPW = G_OI // _NW
_NCOLS = G_OI * N_OUT


def _sc_gather_body(genes_hbm, wtbl_hbm, bri_hbm, cmap_hbm, btbl_hbm,
                    rows_hbm, bg_hbm,
                    idx_v, bri_v, cmap_v, rows_v, brow_v, btile,
                    wsem, bsem):
    wid = lax.axis_index("s") * 2 + lax.axis_index("c")
    base = wid * _GPW
    pltpu.sync_copy(genes_hbm.at[pl.ds(base, _GPW)], idx_v)
    wcopy = pltpu.async_copy(wtbl_hbm.at[idx_v], rows_v, wsem)
    pltpu.sync_copy(bri_hbm.at[pl.ds(base, _GPW)], bri_v)
    bcopy = pltpu.async_copy(btbl_hbm.at[bri_v], brow_v, bsem)
    pltpu.sync_copy(cmap_hbm.at[pl.ds(base, _GPW)], cmap_v)

    bcopy.wait()
    iota16 = lax.iota(jnp.int32, 16)

    def extract_bias(g, carry):
        bbase = cmap_v[g, :] - iota16
        acc = jnp.zeros((16,), dtype=jnp.float32)
        for k in range(8):
            chunk = brow_v[g, pl.ds(k * N_OUT, N_OUT)]
            acc = jnp.where(bbase == k * N_OUT, chunk, acc)
        btile[g, :] = acc
        return carry

    lax.fori_loop(0, _GPW, extract_bias, 0, unroll=False)
    pltpu.sync_copy(btile, bg_hbm.at[pl.ds(base, _GPW)])
    wcopy.wait()
    pltpu.sync_copy(rows_v, rows_hbm.at[pl.ds(base, _GPW)])


def _sc_gather(genes, wtbl256, bri, cmap, btbl128):
    mesh = plsc.VectorSubcoreMesh(core_axis_name="c", subcore_axis_name="s")
    return pl.kernel(
        _sc_gather_body,
        out_type=(
            jax.ShapeDtypeStruct((G_OI, N_HIDDEN * N_OUT), jnp.float32),
            jax.ShapeDtypeStruct((G_OI, N_OUT), jnp.float32),
        ),
        mesh=mesh,
        scratch_types=(
            pltpu.VMEM((_GPW,), jnp.int32),
            pltpu.VMEM((_GPW,), jnp.int32),
            pltpu.VMEM((_GPW, N_OUT), jnp.int32),
            pltpu.VMEM((_GPW, N_HIDDEN * N_OUT), jnp.float32),
            pltpu.VMEM((_GPW, 128), jnp.float32),
            pltpu.VMEM((_GPW, N_OUT), jnp.float32),
            pltpu.SemaphoreType.DMA,
            pltpu.SemaphoreType.DMA,
        ),
    )(genes, wtbl256, bri, cmap, btbl128)


def _tc_transpose_body(rows_ref, bg_ref, wt_ref, b2_ref):
    wt_ref[...] = rows_ref[...].T
    b2_ref[...] = bg_ref[...].T


def _tc_transpose(rows_all, bias_g):
    return pl.pallas_call(
        _tc_transpose_body,
        out_shape=(
            jax.ShapeDtypeStruct((N_HIDDEN * N_OUT, G_OI), jnp.float32),
            jax.ShapeDtypeStruct((N_OUT, G_OI), jnp.float32),
        ),
    )(rows_all, bias_g)


def _tc_matmul_body(latent_ref, w1_ref, b1_ref, scale_ref, shift_ref,
                    wt_ref, bias_ref, out_ref):
    h = jnp.dot(latent_ref[...], w1_ref[...], preferred_element_type=jnp.float32)
    h = jnp.maximum(h + b1_ref[...], 0.0)
    h = h * scale_ref[...] + shift_ref[...]
    out_ref[...] = (
        jnp.dot(h, wt_ref[...], preferred_element_type=jnp.float32)
        + bias_ref[...]
    )


_BB = 256
_GBC = 8192


def _tc_matmul(latent, w1, b1, scale, shift, wt, bias_flat):
    grid = (BATCH // _BB, _NCOLS // _GBC)
    return pl.pallas_call(
        _tc_matmul_body,
        grid=grid,
        in_specs=[
            pl.BlockSpec((_BB, N_LATENT), lambda i, j: (i, 0)),
            pl.BlockSpec((N_LATENT, N_HIDDEN), lambda i, j: (0, 0)),
            pl.BlockSpec((1, N_HIDDEN), lambda i, j: (0, 0)),
            pl.BlockSpec((1, N_HIDDEN), lambda i, j: (0, 0)),
            pl.BlockSpec((1, N_HIDDEN), lambda i, j: (0, 0)),
            pl.BlockSpec((N_HIDDEN, _GBC), lambda i, j: (0, j)),
            pl.BlockSpec((1, _GBC), lambda i, j: (0, j)),
        ],
        out_specs=pl.BlockSpec((_BB, _GBC), lambda i, j: (i, j)),
        out_shape=jax.ShapeDtypeStruct((BATCH, _NCOLS), jnp.float32),
        compiler_params=pltpu.CompilerParams(
            dimension_semantics=("parallel", "parallel"),
        ),
    )(latent, w1, b1, scale, shift, wt, bias_flat)


def kernel(latent, genes_oi, W1, b1, gamma, beta, run_mean, run_var,
           weight_table, bias_table):
    genes = genes_oi.astype(jnp.int32)
    wtbl256 = weight_table.reshape(N_GENES, N_HIDDEN * N_OUT)
    btbl128 = bias_table.reshape(N_GENES * N_OUT // 128, 128)
    bri = genes // 8
    cmap = ((genes % 8) * N_OUT)[:, None] + jnp.arange(N_OUT, dtype=jnp.int32)[None, :]

    rows_all, bias_g = _sc_gather(genes, wtbl256, bri, cmap, btbl128)

    wt_t, b_t = _tc_transpose(rows_all, bias_g)
    wt2 = wt_t.reshape(N_HIDDEN, _NCOLS)
    bias2 = b_t.reshape(1, _NCOLS)

    scale = (gamma / jnp.sqrt(run_var + 1e-5)).reshape(1, N_HIDDEN)
    shift = (beta - run_mean * scale[0]).reshape(1, N_HIDDEN)

    out2 = _tc_matmul(latent, W1, b1.reshape(1, N_HIDDEN), scale, shift,
                      wt2, bias2)
    return out2.reshape(BATCH, N_OUT, G_OI).transpose(0, 2, 1)


# speedup vs baseline: 4.9336x; 1.4376x over previous
import jax
import jax.numpy as jnp
from jax import lax
from jax.experimental import pallas as pl
from jax.experimental.pallas import tpu as pltpu
from jax.experimental.pallas import tpu_sc as plsc

N_LATENT = 128
N_GENES = 100000
N_OUT = 16
N_HIDDEN = 16
BATCH = 1024
G_OI = 2048

_NW = 32
_GPW = G_OI // _NW
_NCOLS = G_OI * N_OUT


def _sc_gather_body(genes_hbm, wtbl_hbm, bri_hbm, cmap_hbm, btbl_hbm,
                    rows_hbm, bg_hbm,
                    idx_v, bri_v, cmap_v, rows_v, brow_v, btile,
                    wsem, bsem):
    wid = lax.axis_index("s") * 2 + lax.axis_index("c")
    base = wid * _GPW
    pltpu.sync_copy(genes_hbm.at[pl.ds(base, _GPW)], idx_v)
    wcopy = pltpu.async_copy(wtbl_hbm.at[idx_v], rows_v, wsem)
    pltpu.sync_copy(bri_hbm.at[pl.ds(base, _GPW)], bri_v)
    bcopy = pltpu.async_copy(btbl_hbm.at[bri_v], brow_v, bsem)
    pltpu.sync_copy(cmap_hbm.at[pl.ds(base, _GPW)], cmap_v)

    bcopy.wait()
    iota16 = lax.iota(jnp.int32, 16)

    def extract_bias(g, carry):
        bbase = cmap_v[g, :] - iota16
        acc = jnp.zeros((16,), dtype=jnp.float32)
        for k in range(8):
            chunk = brow_v[g, pl.ds(k * N_OUT, N_OUT)]
            acc = jnp.where(bbase == k * N_OUT, chunk, acc)
        btile[g, :] = acc
        return carry

    lax.fori_loop(0, _GPW, extract_bias, 0, unroll=False)
    pltpu.sync_copy(btile, bg_hbm.at[pl.ds(base, _GPW)])
    wcopy.wait()
    pltpu.sync_copy(rows_v, rows_hbm.at[pl.ds(base, _GPW)])


def _sc_gather(genes, wtbl256, bri, cmap, btbl128):
    mesh = plsc.VectorSubcoreMesh(core_axis_name="c", subcore_axis_name="s")
    return pl.kernel(
        _sc_gather_body,
        out_type=(
            jax.ShapeDtypeStruct((G_OI, N_HIDDEN * N_OUT), jnp.float32),
            jax.ShapeDtypeStruct((G_OI, N_OUT), jnp.float32),
        ),
        mesh=mesh,
        scratch_types=(
            pltpu.VMEM((_GPW,), jnp.int32),
            pltpu.VMEM((_GPW,), jnp.int32),
            pltpu.VMEM((_GPW, N_OUT), jnp.int32),
            pltpu.VMEM((_GPW, N_HIDDEN * N_OUT), jnp.float32),
            pltpu.VMEM((_GPW, 128), jnp.float32),
            pltpu.VMEM((_GPW, N_OUT), jnp.float32),
            pltpu.SemaphoreType.DMA,
            pltpu.SemaphoreType.DMA,
        ),
    )(genes, wtbl256, bri, cmap, btbl128)


def _tc_transpose_body(rows_ref, bg_ref, wt9_ref, bt_ref):
    at = rows_ref[...].T
    for ot in range(2):
        for h in range(N_HIDDEN):
            s = h * 16 + ot * 8
            wt9_ref[ot, pl.ds(h * 8, 8), :] = at[s:s + 8, :]
    bt_ref[...] = bg_ref[...].T


def _tc_transpose(rows_all, bias_g):
    return pl.pallas_call(
        _tc_transpose_body,
        out_shape=(
            jax.ShapeDtypeStruct((2, 128, G_OI), jnp.float32),
            jax.ShapeDtypeStruct((N_OUT, G_OI), jnp.float32),
        ),
    )(rows_all, bias_g)


_BB = 256
_GT = 4


def _tc_matmul_body(latent_ref, w1_ref, b1_ref, scale_ref, shift_ref,
                    wt9_ref, bias_ref, out_ref):
    h = jnp.dot(latent_ref[...], w1_ref[...], preferred_element_type=jnp.float32)
    h = jnp.maximum(h + b1_ref[...], 0.0)
    h = h * scale_ref[...] + shift_ref[...]
    ii = lax.broadcasted_iota(jnp.int32, (N_HIDDEN, 128), 0)
    kk = lax.broadcasted_iota(jnp.int32, (N_HIDDEN, 128), 1) // 8
    expand = jnp.where(ii == kk, 1.0, 0.0).astype(jnp.float32)
    hrep = jnp.dot(h, expand, preferred_element_type=jnp.float32)
    h3 = jnp.broadcast_to(hrep[:, None, :], (_BB, 8, 128)).reshape(_BB * 8, 128)
    ri = lax.broadcasted_iota(jnp.int32, (_BB * 8, 128), 0) % 8
    ki = lax.broadcasted_iota(jnp.int32, (_BB * 8, 128), 1) % 8
    lhs = jnp.where(ri == ki, h3, 0.0)
    res = jnp.dot(lhs, wt9_ref[0], preferred_element_type=jnp.float32)
    bb = jnp.broadcast_to(bias_ref[0][None], (_BB, 8, _GT * 128))
    res = res + bb.reshape(_BB * 8, _GT * 128)
    for gt in range(_GT):
        out_ref[:, 0, gt] = res[:, gt * 128:(gt + 1) * 128].reshape(_BB, 8, 128)


def _tc_matmul(latent, w1, b1, scale, shift, wt9, bias9):
    grid = (BATCH // _BB, 2, N_OUT // 8 // 2 * G_OI // (128 * _GT))
    return pl.pallas_call(
        _tc_matmul_body,
        grid=grid,
        in_specs=[
            pl.BlockSpec((_BB, N_LATENT), lambda i, j, k: (i, 0)),
            pl.BlockSpec((N_LATENT, N_HIDDEN), lambda i, j, k: (0, 0)),
            pl.BlockSpec((1, N_HIDDEN), lambda i, j, k: (0, 0)),
            pl.BlockSpec((1, N_HIDDEN), lambda i, j, k: (0, 0)),
            pl.BlockSpec((1, N_HIDDEN), lambda i, j, k: (0, 0)),
            pl.BlockSpec((1, 128, _GT * 128), lambda i, j, k: (j, 0, k)),
            pl.BlockSpec((1, 8, _GT * 128), lambda i, j, k: (j, 0, k)),
        ],
        out_specs=pl.BlockSpec((_BB, 1, _GT, 8, 128),
                               lambda i, j, k: (i, j, k, 0, 0)),
        out_shape=jax.ShapeDtypeStruct((BATCH, 2, N_OUT, 8, 128), jnp.float32),
        compiler_params=pltpu.CompilerParams(
            dimension_semantics=("parallel", "parallel", "parallel"),
        ),
    )(latent, w1, b1, scale, shift, wt9, bias9)


def kernel(latent, genes_oi, W1, b1, gamma, beta, run_mean, run_var,
           weight_table, bias_table):
    genes = genes_oi.astype(jnp.int32)
    wtbl256 = weight_table.reshape(N_GENES, N_HIDDEN * N_OUT)
    btbl128 = bias_table.reshape(N_GENES * N_OUT // 128, 128)
    bri = genes // 8
    cmap = ((genes % 8) * N_OUT)[:, None] + jnp.arange(N_OUT, dtype=jnp.int32)[None, :]

    rows_all, bias_g = _sc_gather(genes, wtbl256, bri, cmap, btbl128)

    wt9, bias_t = _tc_transpose(rows_all, bias_g)
    bias9 = bias_t.reshape(2, 8, G_OI)

    scale = (gamma / jnp.sqrt(run_var + 1e-5)).reshape(1, N_HIDDEN)
    shift = (beta - run_mean * scale[0]).reshape(1, N_HIDDEN)

    out5 = _tc_matmul(latent, W1, b1.reshape(1, N_HIDDEN), scale, shift,
                      wt9, bias9)
    out = out5.reshape(BATCH, 2, N_OUT, 8, 128).transpose(0, 2, 4, 1, 3)
    return out.reshape(BATCH, G_OI, N_OUT)
